# Initial kernel scaffold; baseline (speedup 1.0000x reference)
import jax
import jax.numpy as jnp
from jax.experimental import pallas as pl


def kernel(x, points, W1, b1, W2, b2):
    raise NotImplementedError("write your pallas kernel here")


# speedup vs baseline: 3.6147x; 3.6147x over previous
import functools

import jax
import jax.numpy as jnp
from jax import lax
from jax.experimental import pallas as pl
from jax.experimental.pallas import tpu as pltpu
from jax.experimental.pallas import tpu_sc as plsc

K_NB = 16
LANES = 16
NC = 2
NS = 16
NW = NC * NS
BIG_I32 = 2147483647

_HI = jax.lax.Precision.HIGHEST


def _gelu(v):
    return 0.5 * v * (1.0 + lax.erf(v * 0.7071067811865476))


def _prep_body(x_ref, at_ref, ct_ref, b1_ref, y_ref, c_ref):
    xb = x_ref[...]
    y_ref[...] = jnp.dot(xb, at_ref[...], preferred_element_type=jnp.float32,
                         precision=_HI)
    c_ref[...] = jnp.dot(xb, ct_ref[...], preferred_element_type=jnp.float32,
                         precision=_HI) + b1_ref[...]


def _prep(x2, at, ct, b1row):
    n, d = x2.shape
    d2 = at.shape[1]
    return pl.pallas_call(
        _prep_body,
        out_shape=[
            jax.ShapeDtypeStruct((n, d2), jnp.float32),
            jax.ShapeDtypeStruct((n, d2), jnp.float32),
        ],
    )(x2, at, ct, b1row)


def _topk_stage(qpw, nv, w, px_hbm, py_hbm, pz_hbm, pxb_hbm, pyb_hbm,
                pzb_hbm, xs, ys, zs, xb, yb, zb, sq, idxb):
    pltpu.sync_copy(px_hbm, xs)
    pltpu.sync_copy(py_hbm, ys)
    pltpu.sync_copy(pz_hbm, zs)
    pltpu.sync_copy(pxb_hbm, xb)
    pltpu.sync_copy(pyb_hbm, yb)
    pltpu.sync_copy(pzb_hbm, zb)

    def sq_step(s, carry):
        b = s * LANES
        xv = xs[pl.ds(b, LANES)]
        yv = ys[pl.ds(b, LANES)]
        zv = zs[pl.ds(b, LANES)]
        sq[pl.ds(b, LANES)] = xv * xv + yv * yv + zv * zv
        return carry

    lax.fori_loop(0, nv, sq_step, 0)

    iota = lax.broadcasted_iota(jnp.int32, (LANES,), 0)
    inf_v = jnp.full((LANES,), jnp.inf, dtype=jnp.float32)
    zero_i = jnp.zeros((LANES,), jnp.int32)
    big_v = jnp.full((LANES,), BIG_I32, dtype=jnp.int32)

    def group_step(g, carry):
        base = w * qpw + g * LANES
        qxv = xb[pl.ds(base, LANES)]
        qyv = yb[pl.ds(base, LANES)]
        qzv = zb[pl.ds(base, LANES)]
        qsqv = sq[pl.ds(base, LANES)]
        for lane in range(LANES):
            qx = jnp.full((LANES,), qxv[lane])
            qy = jnp.full((LANES,), qyv[lane])
            qz = jnp.full((LANES,), qzv[lane])
            qsq = jnp.full((LANES,), qsqv[lane])

            def key_step(s, tk):
                tv, ti, t15v, rv, ri, zmv = tk
                b = s * LANES
                kx = xb[pl.ds(b, LANES)]
                ky = yb[pl.ds(b, LANES)]
                kz = zb[pl.ds(b, LANES)]
                ksq = sq[pl.ds(b, LANES)]
                dot = qx * kx + qy * ky + qz * kz
                v = (qsq + ksq) - (dot + dot)
                v = jnp.maximum(v, 0.0)
                kidx = iota + b
                zmv = jnp.minimum(zmv, jnp.where(v == 0.0, kidx, big_v))
                pred = jnp.any(v < t15v)

                def do_merge(ops):
                    v_, kidx_, tv_, ti_, rv_, ri_ = ops
                    sv, si = plsc.sort_key_val(v_, kidx_, descending=True)
                    m = sv < tv_
                    tn = jnp.where(m, sv, tv_)
                    tin = jnp.where(m, si, ti_)
                    un = jnp.where(m, tv_, sv)
                    uin = jnp.where(m, ti_, si)
                    ts, tis = plsc.sort_key_val(tn, tin)
                    t15n = jnp.full((LANES,), ts[LANES - 1])
                    m3 = (un < rv_) | ((un == rv_) & (uin < ri_))
                    rn = jnp.where(m3, un, rv_)
                    rin = jnp.where(m3, uin, ri_)
                    return ts, tis, t15n, rn, rin

                def no_merge(ops):
                    v_, kidx_, tv_, ti_, rv_, ri_ = ops
                    m2 = (v_ < rv_) | ((v_ == rv_) & (kidx_ < ri_))
                    rn = jnp.where(m2, v_, rv_)
                    rin = jnp.where(m2, kidx_, ri_)
                    return tv_, ti_, t15v, rn, rin

                tv, ti, t15v, rv, ri = lax.cond(
                    pred, do_merge, no_merge, (v, kidx, tv, ti, rv, ri))
                return tv, ti, t15v, rv, ri, zmv

            _, tis, _, rv, ri, zmv = lax.fori_loop(
                0, nv, key_step,
                (inf_v, zero_i, inf_v, inf_v, zero_i, big_v))
            rs, ris = plsc.sort_key_val(rv, ri)
            rcand = jnp.where(rs == jnp.full((LANES,), rs[0]), ris, big_v)
            rcs, _ = plsc.sort_key_val(rcand, rcand)
            r17i_b = jnp.full((LANES,), rcs[0])
            zms, _ = plsc.sort_key_val(zmv, zmv)
            zmin_b = jnp.full((LANES,), zms[0])
            drop = jnp.where(zmin_b < big_v, tis == zmin_b, iota == 0)
            out_idx = jnp.where(drop, r17i_b, tis)
            idxb[pl.ds((g * LANES + lane) * LANES, LANES)] = out_idx
        return carry

    lax.fori_loop(0, qpw // LANES, group_step, 0)


def _knn_body(n, qpw, nv, px_hbm, py_hbm, pz_hbm, pxb_hbm, pyb_hbm, pzb_hbm,
              y_hbm, gath_hbm, xs, ys, zs, xb, yb, zb, sq, idxb, rowbuf, dsem):
    cid = lax.axis_index("c")
    sid = lax.axis_index("s")
    w = sid * NC + cid
    _topk_stage(qpw, nv, w, px_hbm, py_hbm, pz_hbm, pxb_hbm, pyb_hbm,
                pzb_hbm, xs, ys, zs, xb, yb, zb, sq, idxb)

    rows_per_chunk = rowbuf.shape[0]
    nchunks = (qpw * K_NB) // rows_per_chunk
    for ch in range(nchunks):
        cp = pltpu.async_copy(
            y_hbm.at[idxb.at[pl.ds(ch * rows_per_chunk, rows_per_chunk)]],
            rowbuf, dsem)
        cp.wait()
        pltpu.sync_copy(
            rowbuf,
            gath_hbm.at[pl.ds(w * qpw * K_NB + ch * rows_per_chunk,
                              rows_per_chunk)])


def _knn_gather(px, py, pz, pxb, pyb, pzb, y):
    n = px.shape[0]
    d2 = y.shape[1]
    qpw = n // NW
    nv = n // LANES
    mesh = plsc.VectorSubcoreMesh(core_axis_name="c", subcore_axis_name="s")
    kern = pl.kernel(
        functools.partial(_knn_body, n, qpw, nv),
        out_type=jax.ShapeDtypeStruct((K_NB * n, d2), jnp.float32),
        mesh=mesh,
        compiler_params=pltpu.CompilerParams(needs_layout_passes=False),
        scratch_types=[
            pltpu.VMEM((n,), jnp.float32),
            pltpu.VMEM((n,), jnp.float32),
            pltpu.VMEM((n,), jnp.float32),
            pltpu.VMEM((n,), jnp.float32),
            pltpu.VMEM((n,), jnp.float32),
            pltpu.VMEM((n,), jnp.float32),
            pltpu.VMEM((n,), jnp.float32),
            pltpu.VMEM((K_NB * qpw,), jnp.int32),
            pltpu.VMEM((qpw, d2), jnp.float32),
            pltpu.SemaphoreType.DMA,
        ],
    )
    return kern(px, py, pz, pxb, pyb, pzb, y)


def _mlp_body(bq, gath_ref, c_ref, w2t_ref, b2_ref, out_ref):
    d2 = c_ref.shape[1]
    d = w2t_ref.shape[1]
    g3 = gath_ref[...].reshape(bq, K_NB, d2)
    h1 = _gelu(g3 + c_ref[...][:, None, :])
    h2 = _gelu(jnp.dot(h1.reshape(bq * K_NB, d2), w2t_ref[...],
                       preferred_element_type=jnp.float32,
                       precision=_HI) + b2_ref[...])
    out_ref[...] = jnp.mean(h2.reshape(bq, K_NB, d), axis=1)


def _mlp(gath, c, w2t, b2row):
    n, d2 = c.shape
    d = w2t.shape[1]
    bq = 256
    return pl.pallas_call(
        functools.partial(_mlp_body, bq),
        grid=(n // bq,),
        in_specs=[
            pl.BlockSpec((bq * K_NB, d2), lambda i: (i, 0)),
            pl.BlockSpec((bq, d2), lambda i: (i, 0)),
            pl.BlockSpec((d2, d), lambda i: (0, 0)),
            pl.BlockSpec((1, d), lambda i: (0, 0)),
        ],
        out_specs=pl.BlockSpec((bq, d), lambda i: (i, 0)),
        out_shape=jax.ShapeDtypeStruct((n, d), jnp.float32),
    )(gath, c, w2t, b2row)


def kernel(x, points, W1, b1, W2, b2):
    _, n, d = x.shape
    x2 = x[0]
    p2 = points[0]
    pt = p2.T
    pu = jax.lax.bitcast_convert_type(p2, jnp.uint32)
    pu = ((pu + jnp.uint32(0x7FFF) + ((pu >> 16) & jnp.uint32(1)))
          & jnp.uint32(0xFFFF0000))
    ptb = jax.lax.bitcast_convert_type(pu, jnp.float32).T
    at = W1[:, :d].T
    ct = (W1[:, d:] - W1[:, :d]).T
    y, c = _prep(x2, at, ct, b1.reshape(1, -1))
    gath = _knn_gather(pt[0], pt[1], pt[2], ptb[0], ptb[1], ptb[2], y)
    out = _mlp(gath, c, W2.T, b2.reshape(1, -1))
    return out.reshape(1, n, d)


# speedup vs baseline: 3.9966x; 1.1057x over previous
import functools

import jax
import jax.numpy as jnp
from jax import lax
from jax.experimental import pallas as pl
from jax.experimental.pallas import tpu as pltpu
from jax.experimental.pallas import tpu_sc as plsc

K_NB = 16
LANES = 16
NC = 2
NS = 16
NW = NC * NS
BIG_I32 = 2147483647

_HI = jax.lax.Precision.HIGHEST


def _gelu(v):
    return 0.5 * v * (1.0 + lax.erf(v * 0.7071067811865476))


def _prep_body(x_ref, at_ref, ct_ref, b1_ref, y_ref, c_ref):
    xb = x_ref[...]
    y_ref[...] = jnp.dot(xb, at_ref[...], preferred_element_type=jnp.float32,
                         precision=_HI)
    c_ref[...] = jnp.dot(xb, ct_ref[...], preferred_element_type=jnp.float32,
                         precision=_HI) + b1_ref[...]


def _prep(x2, at, ct, b1row):
    n, d = x2.shape
    d2 = at.shape[1]
    return pl.pallas_call(
        _prep_body,
        out_shape=[
            jax.ShapeDtypeStruct((n, d2), jnp.float32),
            jax.ShapeDtypeStruct((n, d2), jnp.float32),
        ],
    )(x2, at, ct, b1row)


def _topk_stage(qpw, nv, w, px_hbm, py_hbm, pz_hbm, pxb_hbm, pyb_hbm,
                pzb_hbm, xs, ys, zs, xb, yb, zb, sq, idxb):
    pltpu.sync_copy(px_hbm, xs)
    pltpu.sync_copy(py_hbm, ys)
    pltpu.sync_copy(pz_hbm, zs)
    pltpu.sync_copy(pxb_hbm, xb)
    pltpu.sync_copy(pyb_hbm, yb)
    pltpu.sync_copy(pzb_hbm, zb)

    def sq_step(s, carry):
        b = s * LANES
        xv = xs[pl.ds(b, LANES)]
        yv = ys[pl.ds(b, LANES)]
        zv = zs[pl.ds(b, LANES)]
        sq[pl.ds(b, LANES)] = xv * xv + yv * yv + zv * zv
        return carry

    lax.fori_loop(0, nv, sq_step, 0)

    iota = lax.broadcasted_iota(jnp.int32, (LANES,), 0)
    inf_v = jnp.full((LANES,), jnp.inf, dtype=jnp.float32)
    zero_i = jnp.zeros((LANES,), jnp.int32)
    big_v = jnp.full((LANES,), BIG_I32, dtype=jnp.int32)

    def group_step(g, carry):
        base = w * qpw + g * LANES
        qxv = xb[pl.ds(base, LANES)]
        qyv = yb[pl.ds(base, LANES)]
        qzv = zb[pl.ds(base, LANES)]
        qsqv = sq[pl.ds(base, LANES)]
        for lane in range(LANES):
            qx = jnp.full((LANES,), qxv[lane])
            qy = jnp.full((LANES,), qyv[lane])
            qz = jnp.full((LANES,), qzv[lane])
            qsq = jnp.full((LANES,), qsqv[lane])

            def key_step(s, tk):
                tv, ti, t15v, rv, ri = tk
                b = s * LANES
                kx = xb[pl.ds(b, LANES)]
                ky = yb[pl.ds(b, LANES)]
                kz = zb[pl.ds(b, LANES)]
                ksq = sq[pl.ds(b, LANES)]
                dot = qx * kx + qy * ky + qz * kz
                v = (qsq + ksq) - (dot + dot)
                v = jnp.maximum(v, 0.0)
                kidx = iota + b
                cnt = plsc.all_reduce_population_count(v < t15v)
                pred = cnt[0] > 0

                def do_merge(ops):
                    v_, kidx_, tv_, ti_, rv_, ri_ = ops
                    sv, si = plsc.sort_key_val(v_, kidx_, descending=True)
                    m = sv < tv_
                    tn = jnp.where(m, sv, tv_)
                    tin = jnp.where(m, si, ti_)
                    un = jnp.where(m, tv_, sv)
                    uin = jnp.where(m, ti_, si)
                    ts, tis = plsc.sort_key_val(tn, tin)
                    t15n = jnp.full((LANES,), ts[LANES - 1])
                    m3 = un < rv_
                    rn = jnp.where(m3, un, rv_)
                    rin = jnp.where(m3, uin, ri_)
                    return ts, tis, t15n, rn, rin

                def no_merge(ops):
                    v_, kidx_, tv_, ti_, rv_, ri_ = ops
                    m2 = v_ < rv_
                    rn = jnp.where(m2, v_, rv_)
                    rin = jnp.where(m2, kidx_, ri_)
                    return tv_, ti_, t15v, rn, rin

                return lax.cond(
                    pred, do_merge, no_merge, (v, kidx, tv, ti, rv, ri))

            tv, tis, _, rv, ri = lax.fori_loop(
                0, nv, key_step,
                (inf_v, zero_i, inf_v, inf_v, zero_i), unroll=4)
            rs, ris = plsc.sort_key_val(rv, ri)
            rcand = jnp.where(rs == jnp.full((LANES,), rs[0]), ris, big_v)
            rcs, _ = plsc.sort_key_val(rcand, rcand)
            r17i_b = jnp.full((LANES,), rcs[0])
            zc = jnp.where(tv == 0.0, tis, big_v)
            zms, _ = plsc.sort_key_val(zc, zc)
            zmin_b = jnp.full((LANES,), zms[0])
            drop = jnp.where(zmin_b < big_v, tis == zmin_b, iota == 0)
            out_idx = jnp.where(drop, r17i_b, tis)
            idxb[pl.ds((g * LANES + lane) * LANES, LANES)] = out_idx
        return carry

    lax.fori_loop(0, qpw // LANES, group_step, 0)


def _knn_body(n, qpw, nv, px_hbm, py_hbm, pz_hbm, pxb_hbm, pyb_hbm, pzb_hbm,
              y_hbm, gath_hbm, xs, ys, zs, xb, yb, zb, sq, idxb, rowbuf, dsem):
    cid = lax.axis_index("c")
    sid = lax.axis_index("s")
    w = sid * NC + cid
    _topk_stage(qpw, nv, w, px_hbm, py_hbm, pz_hbm, pxb_hbm, pyb_hbm,
                pzb_hbm, xs, ys, zs, xb, yb, zb, sq, idxb)

    rows_per_chunk = rowbuf.shape[0]
    nchunks = (qpw * K_NB) // rows_per_chunk
    for ch in range(nchunks):
        cp = pltpu.async_copy(
            y_hbm.at[idxb.at[pl.ds(ch * rows_per_chunk, rows_per_chunk)]],
            rowbuf, dsem)
        cp.wait()
        pltpu.sync_copy(
            rowbuf,
            gath_hbm.at[pl.ds(w * qpw * K_NB + ch * rows_per_chunk,
                              rows_per_chunk)])


def _knn_gather(px, py, pz, pxb, pyb, pzb, y):
    n = px.shape[0]
    d2 = y.shape[1]
    qpw = n // NW
    nv = n // LANES
    mesh = plsc.VectorSubcoreMesh(core_axis_name="c", subcore_axis_name="s")
    kern = pl.kernel(
        functools.partial(_knn_body, n, qpw, nv),
        out_type=jax.ShapeDtypeStruct((K_NB * n, d2), jnp.float32),
        mesh=mesh,
        compiler_params=pltpu.CompilerParams(needs_layout_passes=False),
        scratch_types=[
            pltpu.VMEM((n,), jnp.float32),
            pltpu.VMEM((n,), jnp.float32),
            pltpu.VMEM((n,), jnp.float32),
            pltpu.VMEM((n,), jnp.float32),
            pltpu.VMEM((n,), jnp.float32),
            pltpu.VMEM((n,), jnp.float32),
            pltpu.VMEM((n,), jnp.float32),
            pltpu.VMEM((K_NB * qpw,), jnp.int32),
            pltpu.VMEM((qpw, d2), jnp.float32),
            pltpu.SemaphoreType.DMA,
        ],
    )
    return kern(px, py, pz, pxb, pyb, pzb, y)


def _mlp_body(bq, gath_ref, c_ref, w2t_ref, b2_ref, out_ref):
    d2 = c_ref.shape[1]
    d = w2t_ref.shape[1]
    g3 = gath_ref[...].reshape(bq, K_NB, d2)
    h1 = _gelu(g3 + c_ref[...][:, None, :])
    h2 = _gelu(jnp.dot(h1.reshape(bq * K_NB, d2), w2t_ref[...],
                       preferred_element_type=jnp.float32,
                       precision=_HI) + b2_ref[...])
    out_ref[...] = jnp.mean(h2.reshape(bq, K_NB, d), axis=1)


def _mlp(gath, c, w2t, b2row):
    n, d2 = c.shape
    d = w2t.shape[1]
    bq = 256
    return pl.pallas_call(
        functools.partial(_mlp_body, bq),
        grid=(n // bq,),
        in_specs=[
            pl.BlockSpec((bq * K_NB, d2), lambda i: (i, 0)),
            pl.BlockSpec((bq, d2), lambda i: (i, 0)),
            pl.BlockSpec((d2, d), lambda i: (0, 0)),
            pl.BlockSpec((1, d), lambda i: (0, 0)),
        ],
        out_specs=pl.BlockSpec((bq, d), lambda i: (i, 0)),
        out_shape=jax.ShapeDtypeStruct((n, d), jnp.float32),
    )(gath, c, w2t, b2row)


def kernel(x, points, W1, b1, W2, b2):
    _, n, d = x.shape
    x2 = x[0]
    p2 = points[0]
    pt = p2.T
    pu = jax.lax.bitcast_convert_type(p2, jnp.uint32)
    pu = ((pu + jnp.uint32(0x7FFF) + ((pu >> 16) & jnp.uint32(1)))
          & jnp.uint32(0xFFFF0000))
    ptb = jax.lax.bitcast_convert_type(pu, jnp.float32).T
    at = W1[:, :d].T
    ct = (W1[:, d:] - W1[:, :d]).T
    y, c = _prep(x2, at, ct, b1.reshape(1, -1))
    gath = _knn_gather(pt[0], pt[1], pt[2], ptb[0], ptb[1], ptb[2], y)
    out = _mlp(gath, c, W2.T, b2.reshape(1, -1))
    return out.reshape(1, n, d)


# speedup vs baseline: 4.4771x; 1.1202x over previous
import functools

import jax
import jax.numpy as jnp
from jax import lax
from jax.experimental import pallas as pl
from jax.experimental.pallas import tpu as pltpu
from jax.experimental.pallas import tpu_sc as plsc

K_NB = 16
LANES = 16
NC = 2
NS = 16
NW = NC * NS
BIG_I32 = 2147483647

_HI = jax.lax.Precision.HIGHEST


def _gelu(v):
    return 0.5 * v * (1.0 + lax.erf(v * 0.7071067811865476))


def _prep_body(x_ref, at_ref, ct_ref, b1_ref, y_ref, c_ref):
    xb = x_ref[...]
    y_ref[...] = jnp.dot(xb, at_ref[...], preferred_element_type=jnp.float32,
                         precision=_HI)
    c_ref[...] = jnp.dot(xb, ct_ref[...], preferred_element_type=jnp.float32,
                         precision=_HI) + b1_ref[...]


def _prep(x2, at, ct, b1row):
    n, d = x2.shape
    d2 = at.shape[1]
    return pl.pallas_call(
        _prep_body,
        out_shape=[
            jax.ShapeDtypeStruct((n, d2), jnp.float32),
            jax.ShapeDtypeStruct((n, d2), jnp.float32),
        ],
    )(x2, at, ct, b1row)


def _topk_stage(qpw, nv, w, px_hbm, py_hbm, pz_hbm, pxb_hbm, pyb_hbm,
                pzb_hbm, xs, ys, zs, xb, yb, zb, sq, idxb):
    pltpu.sync_copy(px_hbm, xs)
    pltpu.sync_copy(py_hbm, ys)
    pltpu.sync_copy(pz_hbm, zs)
    pltpu.sync_copy(pxb_hbm, xb)
    pltpu.sync_copy(pyb_hbm, yb)
    pltpu.sync_copy(pzb_hbm, zb)

    def sq_step(s, carry):
        b = s * LANES
        xv = xs[pl.ds(b, LANES)]
        yv = ys[pl.ds(b, LANES)]
        zv = zs[pl.ds(b, LANES)]
        sq[pl.ds(b, LANES)] = xv * xv + yv * yv + zv * zv
        return carry

    lax.fori_loop(0, nv, sq_step, 0)

    iota = lax.broadcasted_iota(jnp.int32, (LANES,), 0)
    inf_v = jnp.full((LANES,), jnp.inf, dtype=jnp.float32)
    zero_i = jnp.zeros((LANES,), jnp.int32)
    big_v = jnp.full((LANES,), BIG_I32, dtype=jnp.int32)

    CH = 8

    def _merge(v_, kidx_, tv_, ti_, rv_, ri_, t15v):
        sv, si = plsc.sort_key_val(v_, kidx_, descending=True)
        m = sv < tv_
        tn = jnp.where(m, sv, tv_)
        tin = jnp.where(m, si, ti_)
        un = jnp.where(m, tv_, sv)
        uin = jnp.where(m, ti_, si)
        ts, tis = plsc.sort_key_val(tn, tin)
        t15n = jnp.full((LANES,), ts[LANES - 1])
        m3 = un < rv_
        rn = jnp.where(m3, un, rv_)
        rin = jnp.where(m3, uin, ri_)
        return ts, tis, t15n, rn, rin

    def query_step(ql, carry):
        q = w * qpw + ql
        qsplat = jnp.full((LANES,), q, dtype=jnp.int32)
        qx = plsc.load_gather(xb, [qsplat])
        qy = plsc.load_gather(yb, [qsplat])
        qz = plsc.load_gather(zb, [qsplat])
        qsq = plsc.load_gather(sq, [qsplat])

        def chunk_step(s, tk):
            b0 = s * (CH * LANES)
            vs = []
            mch = None
            for j in range(CH):
                b = b0 + j * LANES
                kx = xb[pl.ds(b, LANES)]
                ky = yb[pl.ds(b, LANES)]
                kz = zb[pl.ds(b, LANES)]
                ksq = sq[pl.ds(b, LANES)]
                dot = qx * kx + qy * ky + qz * kz
                v = (qsq + ksq) - (dot + dot)
                v = jnp.maximum(v, 0.0)
                vs.append(v)
                mch = v if mch is None else jnp.minimum(mch, v)
            cnt = plsc.all_reduce_population_count(mch < tk[2])

            def accepted(ops):
                tv, ti, t15v, rv, ri = ops
                for j in range(CH):
                    v_ = vs[j]
                    kidx_ = iota + (b0 + j * LANES)
                    cj = plsc.all_reduce_population_count(v_ < t15v)

                    def mrg(o, v_=v_, kidx_=kidx_):
                        return _merge(v_, kidx_, *o[:2], *o[3:], o[2])

                    def rej(o, v_=v_, kidx_=kidx_):
                        m2 = v_ < o[3]
                        return (o[0], o[1], o[2], jnp.where(m2, v_, o[3]),
                                jnp.where(m2, kidx_, o[4]))

                    tv, ti, t15v, rv, ri = lax.cond(
                        cj[0] > 0, mrg, rej, (tv, ti, t15v, rv, ri))
                return tv, ti, t15v, rv, ri

            def rejected(ops):
                tv, ti, t15v, rv, ri = ops
                for j in range(CH):
                    v_ = vs[j]
                    kidx_ = iota + (b0 + j * LANES)
                    m2 = v_ < rv
                    rv = jnp.where(m2, v_, rv)
                    ri = jnp.where(m2, kidx_, ri)
                return tv, ti, t15v, rv, ri

            return lax.cond(cnt[0] > 0, accepted, rejected, tk)

        tv, tis, _, rv, ri = lax.fori_loop(
            0, nv // CH, chunk_step,
            (inf_v, zero_i, inf_v, inf_v, zero_i))
        rs, ris = plsc.sort_key_val(rv, ri)
        rcand = jnp.where(rs == jnp.full((LANES,), rs[0]), ris, big_v)
        rcs, _ = plsc.sort_key_val(rcand, rcand)
        r17i_b = jnp.full((LANES,), rcs[0])
        zc = jnp.where(tv == 0.0, tis, big_v)
        zms, _ = plsc.sort_key_val(zc, zc)
        zmin_b = jnp.full((LANES,), zms[0])
        drop = jnp.where(zmin_b < big_v, tis == zmin_b, iota == 0)
        out_idx = jnp.where(drop, r17i_b, tis)
        idxb[pl.ds(ql * LANES, LANES)] = out_idx
        return carry

    lax.fori_loop(0, qpw, query_step, 0)


def _knn_body(n, qpw, nv, px_hbm, py_hbm, pz_hbm, pxb_hbm, pyb_hbm, pzb_hbm,
              y_hbm, gath_hbm, xs, ys, zs, xb, yb, zb, sq, idxb, rowbuf, dsem):
    cid = lax.axis_index("c")
    sid = lax.axis_index("s")
    w = sid * NC + cid
    _topk_stage(qpw, nv, w, px_hbm, py_hbm, pz_hbm, pxb_hbm, pyb_hbm,
                pzb_hbm, xs, ys, zs, xb, yb, zb, sq, idxb)

    rows_per_chunk = rowbuf.shape[0]
    nchunks = (qpw * K_NB) // rows_per_chunk
    for ch in range(nchunks):
        cp = pltpu.async_copy(
            y_hbm.at[idxb.at[pl.ds(ch * rows_per_chunk, rows_per_chunk)]],
            rowbuf, dsem)
        cp.wait()
        pltpu.sync_copy(
            rowbuf,
            gath_hbm.at[pl.ds(w * qpw * K_NB + ch * rows_per_chunk,
                              rows_per_chunk)])


def _knn_gather(px, py, pz, pxb, pyb, pzb, y):
    n = px.shape[0]
    d2 = y.shape[1]
    qpw = n // NW
    nv = n // LANES
    mesh = plsc.VectorSubcoreMesh(core_axis_name="c", subcore_axis_name="s")
    kern = pl.kernel(
        functools.partial(_knn_body, n, qpw, nv),
        out_type=jax.ShapeDtypeStruct((K_NB * n, d2), jnp.float32),
        mesh=mesh,
        compiler_params=pltpu.CompilerParams(needs_layout_passes=False),
        scratch_types=[
            pltpu.VMEM((n,), jnp.float32),
            pltpu.VMEM((n,), jnp.float32),
            pltpu.VMEM((n,), jnp.float32),
            pltpu.VMEM((n,), jnp.float32),
            pltpu.VMEM((n,), jnp.float32),
            pltpu.VMEM((n,), jnp.float32),
            pltpu.VMEM((n,), jnp.float32),
            pltpu.VMEM((K_NB * qpw,), jnp.int32),
            pltpu.VMEM((qpw, d2), jnp.float32),
            pltpu.SemaphoreType.DMA,
        ],
    )
    return kern(px, py, pz, pxb, pyb, pzb, y)


def _mlp_body(bq, gath_ref, c_ref, w2t_ref, b2_ref, out_ref):
    d2 = c_ref.shape[1]
    d = w2t_ref.shape[1]
    g3 = gath_ref[...].reshape(bq, K_NB, d2)
    h1 = _gelu(g3 + c_ref[...][:, None, :])
    h2 = _gelu(jnp.dot(h1.reshape(bq * K_NB, d2), w2t_ref[...],
                       preferred_element_type=jnp.float32,
                       precision=_HI) + b2_ref[...])
    out_ref[...] = jnp.mean(h2.reshape(bq, K_NB, d), axis=1)


def _mlp(gath, c, w2t, b2row):
    n, d2 = c.shape
    d = w2t.shape[1]
    bq = 256
    return pl.pallas_call(
        functools.partial(_mlp_body, bq),
        grid=(n // bq,),
        in_specs=[
            pl.BlockSpec((bq * K_NB, d2), lambda i: (i, 0)),
            pl.BlockSpec((bq, d2), lambda i: (i, 0)),
            pl.BlockSpec((d2, d), lambda i: (0, 0)),
            pl.BlockSpec((1, d), lambda i: (0, 0)),
        ],
        out_specs=pl.BlockSpec((bq, d), lambda i: (i, 0)),
        out_shape=jax.ShapeDtypeStruct((n, d), jnp.float32),
    )(gath, c, w2t, b2row)


def kernel(x, points, W1, b1, W2, b2):
    _, n, d = x.shape
    x2 = x[0]
    p2 = points[0]
    pt = p2.T
    pu = jax.lax.bitcast_convert_type(p2, jnp.uint32)
    pu = ((pu + jnp.uint32(0x7FFF) + ((pu >> 16) & jnp.uint32(1)))
          & jnp.uint32(0xFFFF0000))
    ptb = jax.lax.bitcast_convert_type(pu, jnp.float32).T
    at = W1[:, :d].T
    ct = (W1[:, d:] - W1[:, :d]).T
    y, c = _prep(x2, at, ct, b1.reshape(1, -1))
    gath = _knn_gather(pt[0], pt[1], pt[2], ptb[0], ptb[1], ptb[2], y)
    out = _mlp(gath, c, W2.T, b2.reshape(1, -1))
    return out.reshape(1, n, d)


# speedup vs baseline: 4.5210x; 1.0098x over previous
import functools

import jax
import jax.numpy as jnp
from jax import lax
from jax.experimental import pallas as pl
from jax.experimental.pallas import tpu as pltpu
from jax.experimental.pallas import tpu_sc as plsc

K_NB = 16
LANES = 16
NC = 2
NS = 16
NW = NC * NS
BIG_I32 = 2147483647

_HI = jax.lax.Precision.HIGHEST


def _gelu(v):
    return 0.5 * v * (1.0 + lax.erf(v * 0.7071067811865476))


def _prep_body(x_ref, at_ref, ct_ref, b1_ref, y_ref, c_ref):
    xb = x_ref[...]
    y_ref[...] = jnp.dot(xb, at_ref[...], preferred_element_type=jnp.float32,
                         precision=_HI)
    c_ref[...] = jnp.dot(xb, ct_ref[...], preferred_element_type=jnp.float32,
                         precision=_HI) + b1_ref[...]


def _prep(x2, at, ct, b1row):
    n, d = x2.shape
    d2 = at.shape[1]
    return pl.pallas_call(
        _prep_body,
        out_shape=[
            jax.ShapeDtypeStruct((n, d2), jnp.float32),
            jax.ShapeDtypeStruct((n, d2), jnp.float32),
        ],
    )(x2, at, ct, b1row)


def _topk_stage(qpw, nv, w, px_hbm, py_hbm, pz_hbm, pxb_hbm, pyb_hbm,
                pzb_hbm, xs, ys, zs, xb, yb, zb, sq, idxb):
    pltpu.sync_copy(px_hbm, xs)
    pltpu.sync_copy(py_hbm, ys)
    pltpu.sync_copy(pz_hbm, zs)
    pltpu.sync_copy(pxb_hbm, xb)
    pltpu.sync_copy(pyb_hbm, yb)
    pltpu.sync_copy(pzb_hbm, zb)

    def sq_step(s, carry):
        b = s * LANES
        xv = xs[pl.ds(b, LANES)]
        yv = ys[pl.ds(b, LANES)]
        zv = zs[pl.ds(b, LANES)]
        sq[pl.ds(b, LANES)] = xv * xv + yv * yv + zv * zv
        return carry

    lax.fori_loop(0, nv, sq_step, 0)

    iota = lax.broadcasted_iota(jnp.int32, (LANES,), 0)
    inf_v = jnp.full((LANES,), jnp.inf, dtype=jnp.float32)
    zero_i = jnp.zeros((LANES,), jnp.int32)
    big_v = jnp.full((LANES,), BIG_I32, dtype=jnp.int32)

    CH = 8

    def _merge(v_, kidx_, tv_, ti_, rv_, ri_, t15v):
        sv, si = plsc.sort_key_val(v_, kidx_, descending=True)
        m = sv < tv_
        tn = jnp.where(m, sv, tv_)
        tin = jnp.where(m, si, ti_)
        un = jnp.where(m, tv_, sv)
        uin = jnp.where(m, ti_, si)
        ts, tis = plsc.sort_key_val(tn, tin)
        t15n = jnp.full((LANES,), ts[LANES - 1])
        m3 = un < rv_
        rn = jnp.where(m3, un, rv_)
        rin = jnp.where(m3, uin, ri_)
        return ts, tis, t15n, rn, rin

    def query_step(ql, carry):
        q = w * qpw + ql
        qsplat = jnp.full((LANES,), q, dtype=jnp.int32)
        qx = plsc.load_gather(xb, [qsplat])
        qy = plsc.load_gather(yb, [qsplat])
        qz = plsc.load_gather(zb, [qsplat])
        qsq = plsc.load_gather(sq, [qsplat])
        qx2 = qx + qx
        qy2 = qy + qy
        qz2 = qz + qz

        def chunk_step(s, tk):
            b0 = s * (CH * LANES)
            vs = []
            mch = None
            for j in range(CH):
                b = b0 + j * LANES
                kx = xb[pl.ds(b, LANES)]
                ky = yb[pl.ds(b, LANES)]
                kz = zb[pl.ds(b, LANES)]
                ksq = sq[pl.ds(b, LANES)]
                dot2 = qx2 * kx + qy2 * ky + qz2 * kz
                v = (qsq + ksq) - dot2
                v = jnp.maximum(v, 0.0)
                vs.append(v)
                mch = v if mch is None else jnp.minimum(mch, v)
            cnt = plsc.all_reduce_population_count(mch < tk[2])

            def accepted(ops):
                tv, ti, t15v, rv, ri = ops
                for j in range(CH):
                    v_ = vs[j]
                    kidx_ = iota + (b0 + j * LANES)
                    cj = plsc.all_reduce_population_count(v_ < t15v)

                    def mrg(o, v_=v_, kidx_=kidx_):
                        return _merge(v_, kidx_, *o[:2], *o[3:], o[2])

                    def rej(o, v_=v_, kidx_=kidx_):
                        m2 = v_ < o[3]
                        return (o[0], o[1], o[2], jnp.where(m2, v_, o[3]),
                                jnp.where(m2, kidx_, o[4]))

                    tv, ti, t15v, rv, ri = lax.cond(
                        cj[0] > 0, mrg, rej, (tv, ti, t15v, rv, ri))
                return tv, ti, t15v, rv, ri

            def rejected(ops):
                tv, ti, t15v, rv, ri = ops
                for j in range(CH):
                    v_ = vs[j]
                    kidx_ = iota + (b0 + j * LANES)
                    m2 = v_ < rv
                    rv = jnp.where(m2, v_, rv)
                    ri = jnp.where(m2, kidx_, ri)
                return tv, ti, t15v, rv, ri

            return lax.cond(cnt[0] > 0, accepted, rejected, tk)

        tv, tis, _, rv, ri = lax.fori_loop(
            0, nv // CH, chunk_step,
            (inf_v, zero_i, inf_v, inf_v, zero_i), unroll=2)
        rs, ris = plsc.sort_key_val(rv, ri)
        rcand = jnp.where(rs == jnp.full((LANES,), rs[0]), ris, big_v)
        rcs, _ = plsc.sort_key_val(rcand, rcand)
        r17i_b = jnp.full((LANES,), rcs[0])
        zc = jnp.where(tv == 0.0, tis, big_v)
        zms, _ = plsc.sort_key_val(zc, zc)
        zmin_b = jnp.full((LANES,), zms[0])
        drop = jnp.where(zmin_b < big_v, tis == zmin_b, iota == 0)
        out_idx = jnp.where(drop, r17i_b, tis)
        idxb[pl.ds(ql * LANES, LANES)] = out_idx
        return carry

    lax.fori_loop(0, qpw, query_step, 0)


def _knn_body(n, qpw, nv, px_hbm, py_hbm, pz_hbm, pxb_hbm, pyb_hbm, pzb_hbm,
              y_hbm, gath_hbm, xs, ys, zs, xb, yb, zb, sq, idxb, rowbuf, dsem):
    cid = lax.axis_index("c")
    sid = lax.axis_index("s")
    w = sid * NC + cid
    _topk_stage(qpw, nv, w, px_hbm, py_hbm, pz_hbm, pxb_hbm, pyb_hbm,
                pzb_hbm, xs, ys, zs, xb, yb, zb, sq, idxb)

    rows_per_chunk = rowbuf.shape[0]
    nchunks = (qpw * K_NB) // rows_per_chunk
    for ch in range(nchunks):
        cp = pltpu.async_copy(
            y_hbm.at[idxb.at[pl.ds(ch * rows_per_chunk, rows_per_chunk)]],
            rowbuf, dsem)
        cp.wait()
        pltpu.sync_copy(
            rowbuf,
            gath_hbm.at[pl.ds(w * qpw * K_NB + ch * rows_per_chunk,
                              rows_per_chunk)])


def _knn_gather(px, py, pz, pxb, pyb, pzb, y):
    n = px.shape[0]
    d2 = y.shape[1]
    qpw = n // NW
    nv = n // LANES
    mesh = plsc.VectorSubcoreMesh(core_axis_name="c", subcore_axis_name="s")
    kern = pl.kernel(
        functools.partial(_knn_body, n, qpw, nv),
        out_type=jax.ShapeDtypeStruct((K_NB * n, d2), jnp.float32),
        mesh=mesh,
        compiler_params=pltpu.CompilerParams(needs_layout_passes=False),
        scratch_types=[
            pltpu.VMEM((n,), jnp.float32),
            pltpu.VMEM((n,), jnp.float32),
            pltpu.VMEM((n,), jnp.float32),
            pltpu.VMEM((n,), jnp.float32),
            pltpu.VMEM((n,), jnp.float32),
            pltpu.VMEM((n,), jnp.float32),
            pltpu.VMEM((n,), jnp.float32),
            pltpu.VMEM((K_NB * qpw,), jnp.int32),
            pltpu.VMEM((qpw, d2), jnp.float32),
            pltpu.SemaphoreType.DMA,
        ],
    )
    return kern(px, py, pz, pxb, pyb, pzb, y)


def _mlp_body(bq, gath_ref, c_ref, w2t_ref, b2_ref, out_ref):
    d2 = c_ref.shape[1]
    d = w2t_ref.shape[1]
    g3 = gath_ref[...].reshape(bq, K_NB, d2)
    h1 = _gelu(g3 + c_ref[...][:, None, :])
    h2 = _gelu(jnp.dot(h1.reshape(bq * K_NB, d2), w2t_ref[...],
                       preferred_element_type=jnp.float32,
                       precision=_HI) + b2_ref[...])
    out_ref[...] = jnp.mean(h2.reshape(bq, K_NB, d), axis=1)


def _mlp(gath, c, w2t, b2row):
    n, d2 = c.shape
    d = w2t.shape[1]
    bq = 256
    return pl.pallas_call(
        functools.partial(_mlp_body, bq),
        grid=(n // bq,),
        in_specs=[
            pl.BlockSpec((bq * K_NB, d2), lambda i: (i, 0)),
            pl.BlockSpec((bq, d2), lambda i: (i, 0)),
            pl.BlockSpec((d2, d), lambda i: (0, 0)),
            pl.BlockSpec((1, d), lambda i: (0, 0)),
        ],
        out_specs=pl.BlockSpec((bq, d), lambda i: (i, 0)),
        out_shape=jax.ShapeDtypeStruct((n, d), jnp.float32),
    )(gath, c, w2t, b2row)


def kernel(x, points, W1, b1, W2, b2):
    _, n, d = x.shape
    x2 = x[0]
    p2 = points[0]
    pt = p2.T
    pu = jax.lax.bitcast_convert_type(p2, jnp.uint32)
    pu = ((pu + jnp.uint32(0x7FFF) + ((pu >> 16) & jnp.uint32(1)))
          & jnp.uint32(0xFFFF0000))
    ptb = jax.lax.bitcast_convert_type(pu, jnp.float32).T
    at = W1[:, :d].T
    ct = (W1[:, d:] - W1[:, :d]).T
    y, c = _prep(x2, at, ct, b1.reshape(1, -1))
    gath = _knn_gather(pt[0], pt[1], pt[2], ptb[0], ptb[1], ptb[2], y)
    out = _mlp(gath, c, W2.T, b2.reshape(1, -1))
    return out.reshape(1, n, d)


# speedup vs baseline: 4.5824x; 1.0136x over previous
import functools

import jax
import jax.numpy as jnp
from jax import lax
from jax.experimental import pallas as pl
from jax.experimental.pallas import tpu as pltpu
from jax.experimental.pallas import tpu_sc as plsc

K_NB = 16
LANES = 16
NC = 2
NS = 16
NW = NC * NS
BIG_I32 = 2147483647

_HI = jax.lax.Precision.HIGHEST


def _gelu(v):
    return 0.5 * v * (1.0 + lax.erf(v * 0.7071067811865476))


def _prep_body(x_ref, at_ref, ct_ref, b1_ref, y_ref, c_ref):
    xb = x_ref[...]
    y_ref[...] = jnp.dot(xb, at_ref[...], preferred_element_type=jnp.float32,
                         precision=_HI)
    c_ref[...] = jnp.dot(xb, ct_ref[...], preferred_element_type=jnp.float32,
                         precision=_HI) + b1_ref[...]


def _prep(x2, at, ct, b1row):
    n, d = x2.shape
    d2 = at.shape[1]
    return pl.pallas_call(
        _prep_body,
        out_shape=[
            jax.ShapeDtypeStruct((n, d2), jnp.float32),
            jax.ShapeDtypeStruct((n, d2), jnp.float32),
        ],
    )(x2, at, ct, b1row)


def _topk_stage(qpw, nv, w, px_hbm, py_hbm, pz_hbm, pxb_hbm, pyb_hbm,
                pzb_hbm, xs, ys, zs, xb, yb, zb, sq, idxb):
    pltpu.sync_copy(px_hbm, xs)
    pltpu.sync_copy(py_hbm, ys)
    pltpu.sync_copy(pz_hbm, zs)
    pltpu.sync_copy(pxb_hbm, xb)
    pltpu.sync_copy(pyb_hbm, yb)
    pltpu.sync_copy(pzb_hbm, zb)

    def sq_step(s, carry):
        b = s * LANES
        xv = xs[pl.ds(b, LANES)]
        yv = ys[pl.ds(b, LANES)]
        zv = zs[pl.ds(b, LANES)]
        sq[pl.ds(b, LANES)] = xv * xv + yv * yv + zv * zv
        return carry

    lax.fori_loop(0, nv, sq_step, 0)

    iota = lax.broadcasted_iota(jnp.int32, (LANES,), 0)
    inf_v = jnp.full((LANES,), jnp.inf, dtype=jnp.float32)
    zero_i = jnp.zeros((LANES,), jnp.int32)
    big_v = jnp.full((LANES,), BIG_I32, dtype=jnp.int32)

    CH = 8

    def _merge(v_, kidx_, tv_, ti_, rv_, ri_, t15v):
        sv, si = plsc.sort_key_val(v_, kidx_, descending=True)
        m = sv < tv_
        tn = jnp.where(m, sv, tv_)
        tin = jnp.where(m, si, ti_)
        un = jnp.where(m, tv_, sv)
        uin = jnp.where(m, ti_, si)
        ts, tis = plsc.sort_key_val(tn, tin)
        t15n = jnp.full((LANES,), ts[LANES - 1])
        m3 = un < rv_
        rn = jnp.where(m3, un, rv_)
        rin = jnp.where(m3, uin, ri_)
        return ts, tis, t15n, rn, rin

    def query_step(ql, carry):
        q = w * qpw + ql
        qsplat = jnp.full((LANES,), q, dtype=jnp.int32)
        qx = plsc.load_gather(xb, [qsplat])
        qy = plsc.load_gather(yb, [qsplat])
        qz = plsc.load_gather(zb, [qsplat])
        qsq = plsc.load_gather(sq, [qsplat])
        qx2 = qx + qx
        qy2 = qy + qy
        qz2 = qz + qz

        def chunk_step(s, tk):
            b0 = s * (CH * LANES)
            vs = []
            mch = None
            for j in range(CH):
                b = b0 + j * LANES
                kx = xb[pl.ds(b, LANES)]
                ky = yb[pl.ds(b, LANES)]
                kz = zb[pl.ds(b, LANES)]
                ksq = sq[pl.ds(b, LANES)]
                dot2 = qx2 * kx + qy2 * ky + qz2 * kz
                v = (qsq + ksq) - dot2
                v = jnp.maximum(v, 0.0)
                vs.append(v)
                mch = v if mch is None else jnp.minimum(mch, v)
            cnt = plsc.all_reduce_population_count(mch < tk[2])

            def accepted(ops):
                tv, ti, t15v, rv, ri = ops
                for j in range(CH):
                    v_ = vs[j]
                    kidx_ = iota + (b0 + j * LANES)
                    cj = plsc.all_reduce_population_count(v_ < t15v)

                    def mrg(o, v_=v_, kidx_=kidx_):
                        return _merge(v_, kidx_, *o[:2], *o[3:], o[2])

                    def rej(o, v_=v_, kidx_=kidx_):
                        m2 = v_ < o[3]
                        return (o[0], o[1], o[2], jnp.where(m2, v_, o[3]),
                                jnp.where(m2, kidx_, o[4]))

                    tv, ti, t15v, rv, ri = lax.cond(
                        cj[0] > 0, mrg, rej, (tv, ti, t15v, rv, ri))
                return tv, ti, t15v, rv, ri

            def rejected(ops):
                tv, ti, t15v, rv, ri = ops
                for j in range(CH):
                    v_ = vs[j]
                    kidx_ = iota + (b0 + j * LANES)
                    m2 = v_ < rv
                    rv = jnp.where(m2, v_, rv)
                    ri = jnp.where(m2, kidx_, ri)
                return tv, ti, t15v, rv, ri

            return lax.cond(cnt[0] > 0, accepted, rejected, tk)

        tv, tis, _, rv, ri = lax.fori_loop(
            0, nv // CH, chunk_step,
            (inf_v, zero_i, inf_v, inf_v, zero_i), unroll=2)
        rs, ris = plsc.sort_key_val(rv, ri)
        rcand = jnp.where(rs == jnp.full((LANES,), rs[0]), ris, big_v)
        rcs, _ = plsc.sort_key_val(rcand, rcand)
        r17i_b = jnp.full((LANES,), rcs[0])
        zc = jnp.where(tv == 0.0, tis, big_v)
        zms, _ = plsc.sort_key_val(zc, zc)
        zmin_b = jnp.full((LANES,), zms[0])
        drop = jnp.where(zmin_b < big_v, tis == zmin_b, iota == 0)
        out_idx = jnp.where(drop, r17i_b, tis)
        idxb[pl.ds(ql * LANES, LANES)] = out_idx
        return carry

    lax.fori_loop(0, qpw, query_step, 0)


def _knn_body(n, qpw, nv, px_hbm, py_hbm, pz_hbm, pxb_hbm, pyb_hbm, pzb_hbm,
              y_hbm, gath_hbm, xs, ys, zs, xb, yb, zb, sq, idxb, rowbuf,
              rowbuf2, dsem, dsem2):
    cid = lax.axis_index("c")
    sid = lax.axis_index("s")
    w = sid * NC + cid
    _topk_stage(qpw, nv, w, px_hbm, py_hbm, pz_hbm, pxb_hbm, pyb_hbm,
                pzb_hbm, xs, ys, zs, xb, yb, zb, sq, idxb)

    rows_per_chunk = rowbuf.shape[0]
    nchunks = (qpw * K_NB) // rows_per_chunk
    bufs = (rowbuf, rowbuf2)
    sems = (dsem, dsem2)
    cps = [None, None]
    cps[0] = pltpu.async_copy(
        y_hbm.at[idxb.at[pl.ds(0, rows_per_chunk)]], bufs[0], sems[0])
    for ch in range(nchunks):
        cps[ch % 2].wait()
        nxt = ch + 1
        if nxt < nchunks:
            cps[nxt % 2] = pltpu.async_copy(
                y_hbm.at[idxb.at[pl.ds(nxt * rows_per_chunk,
                                       rows_per_chunk)]],
                bufs[nxt % 2], sems[nxt % 2])
        pltpu.sync_copy(
            bufs[ch % 2],
            gath_hbm.at[pl.ds(w * qpw * K_NB + ch * rows_per_chunk,
                              rows_per_chunk)])


def _knn_gather(px, py, pz, pxb, pyb, pzb, y):
    n = px.shape[0]
    d2 = y.shape[1]
    qpw = n // NW
    nv = n // LANES
    mesh = plsc.VectorSubcoreMesh(core_axis_name="c", subcore_axis_name="s")
    kern = pl.kernel(
        functools.partial(_knn_body, n, qpw, nv),
        out_type=jax.ShapeDtypeStruct((K_NB * n, d2), jnp.float32),
        mesh=mesh,
        compiler_params=pltpu.CompilerParams(needs_layout_passes=False),
        scratch_types=[
            pltpu.VMEM((n,), jnp.float32),
            pltpu.VMEM((n,), jnp.float32),
            pltpu.VMEM((n,), jnp.float32),
            pltpu.VMEM((n,), jnp.float32),
            pltpu.VMEM((n,), jnp.float32),
            pltpu.VMEM((n,), jnp.float32),
            pltpu.VMEM((n,), jnp.float32),
            pltpu.VMEM((K_NB * qpw,), jnp.int32),
            pltpu.VMEM((qpw, d2), jnp.float32),
            pltpu.VMEM((qpw, d2), jnp.float32),
            pltpu.SemaphoreType.DMA,
            pltpu.SemaphoreType.DMA,
        ],
    )
    return kern(px, py, pz, pxb, pyb, pzb, y)


def _mlp_body(bq, gath_ref, c_ref, w2t_ref, b2_ref, out_ref):
    d2 = c_ref.shape[1]
    d = w2t_ref.shape[1]
    g3 = gath_ref[...].reshape(bq, K_NB, d2)
    h1 = _gelu(g3 + c_ref[...][:, None, :])
    h2 = _gelu(jnp.dot(h1.reshape(bq * K_NB, d2), w2t_ref[...],
                       preferred_element_type=jnp.float32,
                       precision=_HI) + b2_ref[...])
    out_ref[...] = jnp.mean(h2.reshape(bq, K_NB, d), axis=1)


def _mlp(gath, c, w2t, b2row):
    n, d2 = c.shape
    d = w2t.shape[1]
    bq = 256
    return pl.pallas_call(
        functools.partial(_mlp_body, bq),
        grid=(n // bq,),
        in_specs=[
            pl.BlockSpec((bq * K_NB, d2), lambda i: (i, 0)),
            pl.BlockSpec((bq, d2), lambda i: (i, 0)),
            pl.BlockSpec((d2, d), lambda i: (0, 0)),
            pl.BlockSpec((1, d), lambda i: (0, 0)),
        ],
        out_specs=pl.BlockSpec((bq, d), lambda i: (i, 0)),
        out_shape=jax.ShapeDtypeStruct((n, d), jnp.float32),
    )(gath, c, w2t, b2row)


def kernel(x, points, W1, b1, W2, b2):
    _, n, d = x.shape
    x2 = x[0]
    p2 = points[0]
    pt = p2.T
    pu = jax.lax.bitcast_convert_type(p2, jnp.uint32)
    pu = ((pu + jnp.uint32(0x7FFF) + ((pu >> 16) & jnp.uint32(1)))
          & jnp.uint32(0xFFFF0000))
    ptb = jax.lax.bitcast_convert_type(pu, jnp.float32).T
    at = W1[:, :d].T
    ct = (W1[:, d:] - W1[:, :d]).T
    y, c = _prep(x2, at, ct, b1.reshape(1, -1))
    gath = _knn_gather(pt[0], pt[1], pt[2], ptb[0], ptb[1], ptb[2], y)
    out = _mlp(gath, c, W2.T, b2.reshape(1, -1))
    return out.reshape(1, n, d)


# speedup vs baseline: 4.6032x; 1.0045x over previous
import functools

import jax
import jax.numpy as jnp
from jax import lax
from jax.experimental import pallas as pl
from jax.experimental.pallas import tpu as pltpu
from jax.experimental.pallas import tpu_sc as plsc

K_NB = 16
LANES = 16
NC = 2
NS = 16
NW = NC * NS
BIG_I32 = 2147483647

_HI = jax.lax.Precision.HIGHEST


def _gelu(v):
    return 0.5 * v * (1.0 + lax.erf(v * 0.7071067811865476))


def _prep_body(x_ref, at_ref, ct_ref, b1_ref, y_ref, c_ref):
    xb = x_ref[...]
    y_ref[...] = jnp.dot(xb, at_ref[...], preferred_element_type=jnp.float32,
                         precision=_HI)
    c_ref[...] = jnp.dot(xb, ct_ref[...], preferred_element_type=jnp.float32,
                         precision=_HI) + b1_ref[...]


def _prep(x2, at, ct, b1row):
    n, d = x2.shape
    d2 = at.shape[1]
    return pl.pallas_call(
        _prep_body,
        out_shape=[
            jax.ShapeDtypeStruct((n, d2), jnp.float32),
            jax.ShapeDtypeStruct((n, d2), jnp.float32),
        ],
    )(x2, at, ct, b1row)


def _topk_stage(qpw, nv, w, px_hbm, py_hbm, pz_hbm, pxb_hbm, pyb_hbm,
                pzb_hbm, xs, ys, zs, xb, yb, zb, sq, idxb):
    pltpu.sync_copy(px_hbm, xs)
    pltpu.sync_copy(py_hbm, ys)
    pltpu.sync_copy(pz_hbm, zs)
    pltpu.sync_copy(pxb_hbm, xb)
    pltpu.sync_copy(pyb_hbm, yb)
    pltpu.sync_copy(pzb_hbm, zb)

    def sq_step(s, carry):
        b = s * LANES
        xv = xs[pl.ds(b, LANES)]
        yv = ys[pl.ds(b, LANES)]
        zv = zs[pl.ds(b, LANES)]
        sq[pl.ds(b, LANES)] = xv * xv + yv * yv + zv * zv
        return carry

    lax.fori_loop(0, nv, sq_step, 0)

    iota = lax.broadcasted_iota(jnp.int32, (LANES,), 0)
    inf_v = jnp.full((LANES,), jnp.inf, dtype=jnp.float32)
    zero_i = jnp.zeros((LANES,), jnp.int32)
    big_v = jnp.full((LANES,), BIG_I32, dtype=jnp.int32)

    CH = 8

    def _merge(v_, kidx_, tv_, ti_, rv_, ri_, t15v):
        sv, si = plsc.sort_key_val(v_, kidx_, descending=True)
        m = sv < tv_
        tn = jnp.where(m, sv, tv_)
        tin = jnp.where(m, si, ti_)
        un = jnp.where(m, tv_, sv)
        uin = jnp.where(m, ti_, si)
        ts, tis = plsc.sort_key_val(tn, tin)
        t15n = jnp.full((LANES,), ts[LANES - 1])
        m3 = un < rv_
        rn = jnp.where(m3, un, rv_)
        rin = jnp.where(m3, uin, ri_)
        return ts, tis, t15n, rn, rin

    def _proc_chunk(tk5, vs, mch, b0):
        cnt = plsc.all_reduce_population_count(mch < tk5[2])

        def accepted(ops):
            tv, ti, t15v, rv, ri = ops
            for j in range(CH):
                v_ = vs[j]
                kidx_ = iota + (b0 + j * LANES)
                cj = plsc.all_reduce_population_count(v_ < t15v)

                def mrg(o, v_=v_, kidx_=kidx_):
                    return _merge(v_, kidx_, *o[:2], *o[3:], o[2])

                def rej(o, v_=v_, kidx_=kidx_):
                    m2 = v_ < o[3]
                    return (o[0], o[1], o[2], jnp.where(m2, v_, o[3]),
                            jnp.where(m2, kidx_, o[4]))

                tv, ti, t15v, rv, ri = lax.cond(
                    cj[0] > 0, mrg, rej, (tv, ti, t15v, rv, ri))
            return tv, ti, t15v, rv, ri

        def rejected(ops):
            tv, ti, t15v, rv, ri = ops
            for j in range(CH):
                v_ = vs[j]
                kidx_ = iota + (b0 + j * LANES)
                m2 = v_ < rv
                rv = jnp.where(m2, v_, rv)
                ri = jnp.where(m2, kidx_, ri)
            return tv, ti, t15v, rv, ri

        return lax.cond(cnt[0] > 0, accepted, rejected, tk5)

    def _qbcast(q):
        qsplat = jnp.full((LANES,), q, dtype=jnp.int32)
        qx = plsc.load_gather(xb, [qsplat])
        qy = plsc.load_gather(yb, [qsplat])
        qz = plsc.load_gather(zb, [qsplat])
        qsq = plsc.load_gather(sq, [qsplat])
        return qx + qx, qy + qy, qz + qz, qsq

    def _finalize(tk5, ql):
        tv, tis, _, rv, ri = tk5
        rs, ris = plsc.sort_key_val(rv, ri)
        rcand = jnp.where(rs == jnp.full((LANES,), rs[0]), ris, big_v)
        rcs, _ = plsc.sort_key_val(rcand, rcand)
        r17i_b = jnp.full((LANES,), rcs[0])
        zc = jnp.where(tv == 0.0, tis, big_v)
        zms, _ = plsc.sort_key_val(zc, zc)
        zmin_b = jnp.full((LANES,), zms[0])
        drop = jnp.where(zmin_b < big_v, tis == zmin_b, iota == 0)
        out_idx = jnp.where(drop, r17i_b, tis)
        idxb[pl.ds(ql * LANES, LANES)] = out_idx

    def pair_step(qp, carry):
        qla = qp * 2
        qlb = qla + 1
        qx2a, qy2a, qz2a, qsqa = _qbcast(w * qpw + qla)
        qx2b, qy2b, qz2b, qsqb = _qbcast(w * qpw + qlb)

        def chunk_step(s, tk):
            b0 = s * (CH * LANES)
            vsa, vsb = [], []
            ma = mb = None
            for j in range(CH):
                b = b0 + j * LANES
                kx = xb[pl.ds(b, LANES)]
                ky = yb[pl.ds(b, LANES)]
                kz = zb[pl.ds(b, LANES)]
                ksq = sq[pl.ds(b, LANES)]
                va = jnp.maximum(
                    (qsqa + ksq) - (qx2a * kx + qy2a * ky + qz2a * kz), 0.0)
                vb = jnp.maximum(
                    (qsqb + ksq) - (qx2b * kx + qy2b * ky + qz2b * kz), 0.0)
                vsa.append(va)
                vsb.append(vb)
                ma = va if ma is None else jnp.minimum(ma, va)
                mb = vb if mb is None else jnp.minimum(mb, vb)
            tka = _proc_chunk(tk[:5], vsa, ma, b0)
            tkb = _proc_chunk(tk[5:], vsb, mb, b0)
            return (*tka, *tkb)

        init5 = (inf_v, zero_i, inf_v, inf_v, zero_i)
        tk = lax.fori_loop(0, nv // CH, chunk_step, (*init5, *init5))
        _finalize(tk[:5], qla)
        _finalize(tk[5:], qlb)
        return carry

    lax.fori_loop(0, qpw // 2, pair_step, 0)


def _knn_body(n, qpw, nv, px_hbm, py_hbm, pz_hbm, pxb_hbm, pyb_hbm, pzb_hbm,
              y_hbm, gath_hbm, xs, ys, zs, xb, yb, zb, sq, idxb, rowbuf,
              rowbuf2, dsem, dsem2):
    cid = lax.axis_index("c")
    sid = lax.axis_index("s")
    w = sid * NC + cid
    _topk_stage(qpw, nv, w, px_hbm, py_hbm, pz_hbm, pxb_hbm, pyb_hbm,
                pzb_hbm, xs, ys, zs, xb, yb, zb, sq, idxb)

    rows_per_chunk = rowbuf.shape[0]
    nchunks = (qpw * K_NB) // rows_per_chunk
    bufs = (rowbuf, rowbuf2)
    sems = (dsem, dsem2)
    cps = [None, None]
    cps[0] = pltpu.async_copy(
        y_hbm.at[idxb.at[pl.ds(0, rows_per_chunk)]], bufs[0], sems[0])
    for ch in range(nchunks):
        cps[ch % 2].wait()
        nxt = ch + 1
        if nxt < nchunks:
            cps[nxt % 2] = pltpu.async_copy(
                y_hbm.at[idxb.at[pl.ds(nxt * rows_per_chunk,
                                       rows_per_chunk)]],
                bufs[nxt % 2], sems[nxt % 2])
        pltpu.sync_copy(
            bufs[ch % 2],
            gath_hbm.at[pl.ds(w * qpw * K_NB + ch * rows_per_chunk,
                              rows_per_chunk)])


def _knn_gather(px, py, pz, pxb, pyb, pzb, y):
    n = px.shape[0]
    d2 = y.shape[1]
    qpw = n // NW
    nv = n // LANES
    mesh = plsc.VectorSubcoreMesh(core_axis_name="c", subcore_axis_name="s")
    kern = pl.kernel(
        functools.partial(_knn_body, n, qpw, nv),
        out_type=jax.ShapeDtypeStruct((K_NB * n, d2), jnp.float32),
        mesh=mesh,
        compiler_params=pltpu.CompilerParams(needs_layout_passes=False),
        scratch_types=[
            pltpu.VMEM((n,), jnp.float32),
            pltpu.VMEM((n,), jnp.float32),
            pltpu.VMEM((n,), jnp.float32),
            pltpu.VMEM((n,), jnp.float32),
            pltpu.VMEM((n,), jnp.float32),
            pltpu.VMEM((n,), jnp.float32),
            pltpu.VMEM((n,), jnp.float32),
            pltpu.VMEM((K_NB * qpw,), jnp.int32),
            pltpu.VMEM((qpw, d2), jnp.float32),
            pltpu.VMEM((qpw, d2), jnp.float32),
            pltpu.SemaphoreType.DMA,
            pltpu.SemaphoreType.DMA,
        ],
    )
    return kern(px, py, pz, pxb, pyb, pzb, y)


def _mlp_body(bq, gath_ref, c_ref, w2t_ref, b2_ref, out_ref):
    d2 = c_ref.shape[1]
    d = w2t_ref.shape[1]
    g3 = gath_ref[...].reshape(bq, K_NB, d2)
    h1 = _gelu(g3 + c_ref[...][:, None, :])
    h2 = _gelu(jnp.dot(h1.reshape(bq * K_NB, d2), w2t_ref[...],
                       preferred_element_type=jnp.float32,
                       precision=_HI) + b2_ref[...])
    out_ref[...] = jnp.mean(h2.reshape(bq, K_NB, d), axis=1)


def _mlp(gath, c, w2t, b2row):
    n, d2 = c.shape
    d = w2t.shape[1]
    bq = 256
    return pl.pallas_call(
        functools.partial(_mlp_body, bq),
        grid=(n // bq,),
        in_specs=[
            pl.BlockSpec((bq * K_NB, d2), lambda i: (i, 0)),
            pl.BlockSpec((bq, d2), lambda i: (i, 0)),
            pl.BlockSpec((d2, d), lambda i: (0, 0)),
            pl.BlockSpec((1, d), lambda i: (0, 0)),
        ],
        out_specs=pl.BlockSpec((bq, d), lambda i: (i, 0)),
        out_shape=jax.ShapeDtypeStruct((n, d), jnp.float32),
    )(gath, c, w2t, b2row)


def kernel(x, points, W1, b1, W2, b2):
    _, n, d = x.shape
    x2 = x[0]
    p2 = points[0]
    pt = p2.T
    pu = jax.lax.bitcast_convert_type(p2, jnp.uint32)
    pu = ((pu + jnp.uint32(0x7FFF) + ((pu >> 16) & jnp.uint32(1)))
          & jnp.uint32(0xFFFF0000))
    ptb = jax.lax.bitcast_convert_type(pu, jnp.float32).T
    at = W1[:, :d].T
    ct = (W1[:, d:] - W1[:, :d]).T
    y, c = _prep(x2, at, ct, b1.reshape(1, -1))
    gath = _knn_gather(pt[0], pt[1], pt[2], ptb[0], ptb[1], ptb[2], y)
    out = _mlp(gath, c, W2.T, b2.reshape(1, -1))
    return out.reshape(1, n, d)
